# baseline (speedup 1.0000x reference)
import jax
import jax.numpy as jnp
from jax import lax
from jax.experimental import pallas as pl
from jax.experimental.pallas import tpu as pltpu
from jax.experimental.pallas import tpu_sc as plsc

B = 16384
VOCAB = 100000
D = 16
NF = 26
ND = 13
K = 8
NCOL = ND + NF

NC = 2
NS = 16
NW = NC * NS
BPW = B // NW

NFP = 32
NTC = NFP // 8
GSZ = 128
EPC = 8 * NFP
GPC = EPC // GSZ
NCHUNK = BPW // 8
EPW = BPW * NFP
TPC = EPC // 64
TPW = EPW // 64
EMBC_LANES = 128


def _sc_gather_body(sp_hbm, table_hbm, out_hbm, sp_v, offs_v, idx0_v,
                    idx1_v, off0_v, off1_v, rows0_v, rows1_v, st0_v, st1_v,
                    sem0, sem1):
    wid = lax.axis_index("s") * NC + lax.axis_index("c")

    pltpu.sync_copy(sp_hbm.at[pl.ds(wid * EPW, EPW)], sp_v)

    iota = lax.broadcasted_iota(jnp.int32, (16,), 0)
    lane_f = iota % 8
    for tc in range(NTC):
        f = lane_f + 8 * tc
        offs_v[pl.ds(tc * 16, 16)] = jnp.where(f < NF, f, f - 8) * VOCAB

    idxb = (idx0_v, idx1_v)
    offb = (off0_v, off1_v)
    rowsb = (rows0_v, rows1_v)
    stb = (st0_v, st1_v)

    def build_idx(c, buf):
        idx_v, off_v = idxb[buf], offb[buf]

        def _u_body(u, _):
            s = sp_v[pl.ds(c * EPC + u * 16, 16)]
            o = offs_v[pl.ds((u // 4) * 16, 16)]
            flat = s + o
            idx_v[u // 8, pl.ds((u % 8) * 16, 16)] = lax.shift_right_logical(
                flat, 3)
            off_v[pl.ds(u * 16, 16)] = lax.shift_left(
                jnp.bitwise_and(flat, 7), 4)
            return 0

        lax.fori_loop(0, EPC // 16, _u_body, 0)

    def fire(c, buf, sem):
        idx_v = idxb[buf]
        for g in range(GPC):
            pltpu.async_copy(table_hbm.at[idx_v.at[g]],
                             rowsb[buf].at[pl.ds(g * GSZ, GSZ)], sem)

    def extract(buf):
        off_v, rows_v, st_v = offb[buf], rowsb[buf], stb[buf]

        def _vg_body(vg, _):
            e = vg * 16 + iota
            colbase = off_v[pl.ds(vg * 16, 16)]
            t = lax.shift_right_logical(e, 6)
            r = jnp.bitwise_and(lax.shift_right_logical(e, 3), 7)
            c0 = lax.shift_left(jnp.bitwise_and(e, 7), 4)
            for d in range(D):
                vals = plsc.load_gather(rows_v, [e, colbase + d])
                plsc.store_scatter(st_v, [t, r, c0 + d], vals)
            return 0

        lax.fori_loop(0, EPC // 16, _vg_body, 0)

    def drain(c, buf, sem):
        idx_v = idxb[buf]
        for g in range(GPC):
            pltpu.make_async_copy(table_hbm.at[idx_v.at[g]],
                                  rowsb[buf].at[pl.ds(g * GSZ, GSZ)],
                                  sem).wait()
        extract(buf)
        pltpu.sync_copy(stb[buf], out_hbm.at[pl.ds(wid * TPW + c * TPC, TPC)])

    def stage(c, buf, sem):
        build_idx(c, buf)
        fire(c, buf, sem)

    stage(0, 0, sem0)

    def _pipe_body(i, _):
        stage(2 * i + 1, 1, sem1)
        drain(2 * i, 0, sem0)
        stage(2 * i + 2, 0, sem0)
        drain(2 * i + 1, 1, sem1)
        return 0

    lax.fori_loop(0, NCHUNK // 2 - 1, _pipe_body, 0)
    stage(NCHUNK - 1, 1, sem1)
    drain(NCHUNK - 2, 0, sem0)
    drain(NCHUNK - 1, 1, sem1)


def _sc_gather(sp_tiled, table8):
    mesh = plsc.VectorSubcoreMesh(core_axis_name="c", subcore_axis_name="s")
    return pl.kernel(
        _sc_gather_body,
        out_type=jax.ShapeDtypeStruct((B // 2, 8, EMBC_LANES), jnp.float32),
        mesh=mesh,
        compiler_params=pltpu.CompilerParams(use_tc_tiling_on_sc=True,
                                            needs_layout_passes=False),
        scratch_types=[
            pltpu.VMEM((EPW,), jnp.int32),
            pltpu.VMEM((NTC * 16,), jnp.int32),
            pltpu.VMEM((GPC, GSZ), jnp.int32),
            pltpu.VMEM((GPC, GSZ), jnp.int32),
            pltpu.VMEM((EPC,), jnp.int32),
            pltpu.VMEM((EPC,), jnp.int32),
            pltpu.VMEM((EPC, 128), jnp.float32),
            pltpu.VMEM((EPC, 128), jnp.float32),
            pltpu.VMEM((TPC, 8, 128), jnp.float32),
            pltpu.VMEM((TPC, 8, 128), jnp.float32),
            pltpu.SemaphoreType.DMA,
            pltpu.SemaphoreType.DMA,
        ],
    )(sp_tiled, table8)


def _dense_body(dense_ref, emb_ref, w0_ref, wd_ref, ws_ref, vd_ref, ve_ref,
                w1_ref, b1_ref, w2_ref, b2_ref, w3_ref, b3_ref, w4_ref,
                b4_ref, w5_ref, b5_ref, out_ref):
    f32 = jnp.float32
    dense = dense_ref[...]
    bb = dense.shape[0]
    et = emb_ref[...].reshape(bb // 8, NTC, 8, EMBC_LANES)
    embs = [et[:, tc].reshape(bb, EMBC_LANES) for tc in range(NTC)]

    def msum(xs, ws):
        acc = jnp.dot(xs[0], ws[0], preferred_element_type=f32)
        for x, wt in zip(xs[1:], ws[1:]):
            acc = acc + jnp.dot(x, wt, preferred_element_type=f32)
        return acc

    ws_b = [ws_ref[pl.ds(tc * EMBC_LANES, EMBC_LANES), :] for tc in range(NTC)]
    ve_b = [ve_ref[pl.ds(tc * EMBC_LANES, EMBC_LANES), :] for tc in range(NTC)]
    w1_b = [w1_ref[pl.ds(tc * EMBC_LANES, EMBC_LANES), :] for tc in range(NTC)]

    lin = jnp.dot(dense, wd_ref[...], preferred_element_type=f32) \
        + msum(embs, ws_b)

    vd = vd_ref[...]
    p = jnp.dot(dense, vd, preferred_element_type=f32) + msum(embs, ve_b)
    q = (jnp.dot(dense * dense, vd * vd, preferred_element_type=f32)
         + msum([e * e for e in embs], [v * v for v in ve_b]))
    inter = 0.5 * jnp.sum(p * p - q, axis=1, keepdims=True)

    h = jnp.maximum(msum(embs, w1_b) + b1_ref[...], 0.0)
    h = jnp.maximum(jnp.dot(h, w2_ref[...], preferred_element_type=f32)
                    + b2_ref[...], 0.0)
    h = jnp.maximum(jnp.dot(h, w3_ref[...], preferred_element_type=f32)
                    + b3_ref[...], 0.0)
    w45 = jnp.dot(w4_ref[...], w5_ref[...], preferred_element_type=f32)
    c45 = jnp.dot(b4_ref[...], w5_ref[...], preferred_element_type=f32) \
        + b5_ref[...]
    deep = jnp.dot(h, w45, preferred_element_type=f32) + c45

    logit = 0.5 * (lin + w0_ref[0, 0] + inter + deep)
    out_ref[...] = 1.0 / (1.0 + jnp.exp(-logit))


def _dense_stage(dense, emb5, w0, wd, wsp, vd, vep, W1p, b1, W2, b2, W3, b3,
                 W4, b4, W5, b5):
    BB = 1024
    grid = (B // BB,)

    def full_spec(a):
        return pl.BlockSpec(a.shape, lambda i: (0,) * a.ndim)

    tail = (w0, wd, wsp, vd, vep, W1p, b1, W2, b2, W3, b3, W4, b4, W5, b5)
    return pl.pallas_call(
        _dense_body,
        grid=grid,
        in_specs=[pl.BlockSpec((BB, ND), lambda i: (i, 0)),
                  pl.BlockSpec((BB // 8 * NTC, 8, EMBC_LANES),
                               lambda i: (i, 0, 0))] +
                 [full_spec(a) for a in tail],
        out_specs=pl.BlockSpec((BB, 1), lambda i: (i, 0)),
        out_shape=jax.ShapeDtypeStruct((B, 1), jnp.float32),
    )(dense, emb5, *tail)


def kernel(inputs, tables, w0, w, V, W1, b1, W2, b2, W3, b3, W4, b4, W5, b5):
    sp_pad = jnp.pad(inputs[:, ND:], ((0, 0), (0, NFP - NF)))
    sp_tiled = sp_pad.reshape(B // 8, 8, NTC, 8).transpose(0, 2, 1, 3) \
        .reshape(B * NFP)
    table8 = tables.reshape(NF * VOCAB // 8, 8 * D)
    emb5 = _sc_gather(sp_tiled, table8)

    dense = inputs[:, :ND].astype(jnp.float32)
    pad = NFP * D - NF * D
    wsp = jnp.pad(w[ND:], ((0, pad), (0, 0)))
    vep = jnp.pad(V[ND:], ((0, pad), (0, 0)))
    W1p = jnp.pad(W1, ((0, pad), (0, 0)))
    return _dense_stage(
        dense, emb5, w0.reshape(1, 1), w[:ND], wsp, V[:ND], vep,
        W1p, b1.reshape(1, 256), W2, b2.reshape(1, 128), W3, b3.reshape(1, 64),
        W4, b4.reshape(1, 64), W5, b5.reshape(1, 1))


# speedup vs baseline: 1.6574x; 1.6574x over previous
import jax
import jax.numpy as jnp
from jax import lax
from jax.experimental import pallas as pl
from jax.experimental.pallas import tpu as pltpu
from jax.experimental.pallas import tpu_sc as plsc

B = 16384
VOCAB = 100000
D = 16
NF = 26
ND = 13
K = 8

NC = 2
NS = 16
NW = NC * NS
BPW = B // NW
EPW = BPW * NF
GSZ = 128
NGRP = EPW // GSZ
GPC = 13
CHUNK = GPC * GSZ
NCHUNK = NGRP // GPC


def _sc_gather_body(sp_hbm, table_hbm, out_hbm, sp_v, offs_v, idx_v, rows_v,
                    sem0, sem1):
    wid = lax.axis_index("s") * NC + lax.axis_index("c")
    ebase = wid * EPW

    pltpu.sync_copy(sp_hbm.at[pl.ds(ebase, EPW)], sp_v)

    iota = lax.broadcasted_iota(jnp.int32, (16,), 0)

    def _offs_body(u, _):
        e = iota + u * 16
        offs_v[pl.ds(u * 16, 16)] = (e % NF) * VOCAB
        return 0

    lax.fori_loop(0, CHUNK // 16, _offs_body, 0)

    def build_idx(c, buf):
        def _idx_body(g, _):
            for l in range(GSZ // 16):
                s = sp_v[pl.ds(c * CHUNK + g * GSZ + l * 16, 16)]
                o = offs_v[pl.ds(g * GSZ + l * 16, 16)]
                idx_v[buf, g, pl.ds(l * 16, 16)] = s + o
            return 0
        lax.fori_loop(0, GPC, _idx_body, 0)

    def fire(c, buf, sem):
        cps = []
        for g in range(GPC):
            cps.append(pltpu.async_copy(
                table_hbm.at[idx_v.at[buf, g]],
                rows_v.at[buf, pl.ds(g * GSZ, GSZ)], sem))
        return cps

    def drain(cps, c, buf):
        for cp in cps:
            cp.wait()
        pltpu.sync_copy(rows_v.at[buf],
                        out_hbm.at[pl.ds(ebase + c * CHUNK, CHUNK)])

    sems = (sem0, sem1)
    build_idx(0, 0)
    inflight = fire(0, 0, sems[0])
    for c in range(1, NCHUNK):
        buf = c % 2
        build_idx(c, buf)
        nxt = fire(c, buf, sems[buf])
        drain(inflight, c - 1, (c - 1) % 2)
        inflight = nxt
    drain(inflight, NCHUNK - 1, (NCHUNK - 1) % 2)


def _sc_gather(sp_flat, table_flat):
    mesh = plsc.VectorSubcoreMesh(core_axis_name="c", subcore_axis_name="s")
    return pl.kernel(
        _sc_gather_body,
        out_type=jax.ShapeDtypeStruct((B * NF, D), jnp.float32),
        mesh=mesh,
        compiler_params=pltpu.CompilerParams(use_tc_tiling_on_sc=False),
        scratch_types=[
            pltpu.VMEM((EPW,), jnp.int32),
            pltpu.VMEM((CHUNK,), jnp.int32),
            pltpu.VMEM((2, GPC, GSZ), jnp.int32),
            pltpu.VMEM((2, CHUNK, D), jnp.float32),
            pltpu.SemaphoreType.DMA,
            pltpu.SemaphoreType.DMA,
        ],
    )(sp_flat, table_flat)


def _dense_body(dense_ref, emb_ref, w0_ref, wd_ref, ws_ref, vd_ref, ve_ref,
                w1_ref, b1_ref, w2_ref, b2_ref, w3_ref, b3_ref, w4_ref,
                b4_ref, w5_ref, b5_ref, out_ref):
    f32 = jnp.float32
    dense = dense_ref[...]
    emb = emb_ref[...]

    lin = (jnp.dot(dense, wd_ref[...], preferred_element_type=f32)
           + jnp.dot(emb, ws_ref[...], preferred_element_type=f32))

    vd = vd_ref[...]
    ve = ve_ref[...]
    p = (jnp.dot(dense, vd, preferred_element_type=f32)
         + jnp.dot(emb, ve, preferred_element_type=f32))
    q = (jnp.dot(dense * dense, vd * vd, preferred_element_type=f32)
         + jnp.dot(emb * emb, ve * ve, preferred_element_type=f32))
    inter = 0.5 * jnp.sum(p * p - q, axis=1, keepdims=True)

    h = jnp.maximum(jnp.dot(emb, w1_ref[...], preferred_element_type=f32)
                    + b1_ref[...], 0.0)
    h = jnp.maximum(jnp.dot(h, w2_ref[...], preferred_element_type=f32)
                    + b2_ref[...], 0.0)
    h = jnp.maximum(jnp.dot(h, w3_ref[...], preferred_element_type=f32)
                    + b3_ref[...], 0.0)
    w45 = jnp.dot(w4_ref[...], w5_ref[...], preferred_element_type=f32)
    c45 = jnp.dot(b4_ref[...], w5_ref[...], preferred_element_type=f32) \
        + b5_ref[...]
    deep = jnp.dot(h, w45, preferred_element_type=f32) + c45

    logit = 0.5 * (lin + w0_ref[0, 0] + inter + deep)
    out_ref[...] = 1.0 / (1.0 + jnp.exp(-logit))


def _dense_stage(dense, emb, w0, wd, ws, vd, ve, W1, b1, W2, b2, W3, b3,
                 W4, b4, W5, b5):
    BB = 1024
    grid = (B // BB,)

    def batch_spec(cols):
        return pl.BlockSpec((BB, cols), lambda i: (i, 0))

    def full_spec(a):
        return pl.BlockSpec(a.shape, lambda i: (0,) * a.ndim)

    return pl.pallas_call(
        _dense_body,
        grid=grid,
        in_specs=[
            batch_spec(ND), batch_spec(NF * D),
            full_spec(w0), full_spec(wd), full_spec(ws), full_spec(vd),
            full_spec(ve), full_spec(W1), full_spec(b1), full_spec(W2),
            full_spec(b2), full_spec(W3), full_spec(b3), full_spec(W4),
            full_spec(b4), full_spec(W5), full_spec(b5),
        ],
        out_specs=batch_spec(1),
        out_shape=jax.ShapeDtypeStruct((B, 1), jnp.float32),
    )(dense, emb, w0, wd, ws, vd, ve, W1, b1, W2, b2, W3, b3, W4, b4, W5, b5)


def kernel(inputs, tables, w0, w, V, W1, b1, W2, b2, W3, b3, W4, b4, W5, b5):
    sp_flat = inputs[:, ND:].reshape(B * NF)
    table_flat = tables.reshape(NF * VOCAB, D)
    emb = _sc_gather(sp_flat, table_flat).reshape(B, NF * D)

    dense = inputs[:, :ND].astype(jnp.float32)
    return _dense_stage(
        dense, emb, w0.reshape(1, 1), w[:ND], w[ND:], V[:ND], V[ND:],
        W1, b1.reshape(1, 256), W2, b2.reshape(1, 128), W3, b3.reshape(1, 64),
        W4, b4.reshape(1, 64), W5, b5.reshape(1, 1))
